# Initial kernel scaffold; baseline (speedup 1.0000x reference)
import jax
import jax.numpy as jnp
from jax.experimental import pallas as pl


def kernel(adj_indices, adj_values, input, M, W):
    raise NotImplementedError("write your pallas kernel here")


# speedup vs baseline: 5.5952x; 5.5952x over previous
import jax
import jax.numpy as jnp
from jax import lax
from jax.experimental import pallas as pl
from jax.experimental.pallas import tpu as pltpu
from jax.experimental.pallas import tpu_sc as plsc

_T, _N, _E, _D = 4, 10000, 320000, 128
_NC, _NS, _L = 2, 16, 16
_EPW = _E // _NS
_K = 128
_IB = 1280
_NIB = _EPW // _IB
_IBT = _EPW - _NIB * _IB
_TCH = _IBT // _K
_TAIL = _IBT - _TCH * _K
_RPW = 632
_RPW_LAST = _N - _RPW * (_NS - 1)


def _bcast_lane(vec16, l):
    idx = jnp.full((_L, 1), l, jnp.int32)
    dn = lax.GatherDimensionNumbers(offset_dims=(), collapsed_slice_dims=(0,),
                                    start_index_map=(0,))
    return lax.gather(vec16, idx, dn, (1,),
                      mode=lax.GatherScatterMode.PROMISE_IN_BOUNDS)


def _mm_body(x_ref, w_ref, o_ref):
    o_ref[...] = jnp.dot(x_ref[...], w_ref[...],
                         preferred_element_type=jnp.float32)


def _xw_matmul(x_flat, W):
    BN = 2000
    return pl.pallas_call(
        _mm_body,
        grid=(x_flat.shape[0] // BN,),
        in_specs=[
            pl.BlockSpec((BN, _D), lambda i: (i, 0)),
            pl.BlockSpec((_D, _D), lambda i: (0, 0)),
        ],
        out_specs=pl.BlockSpec((BN, _D), lambda i: (i, 0)),
        out_shape=jax.ShapeDtypeStruct((x_flat.shape[0], _D), jnp.float32),
    )(x_flat, W)


def _spmm_body(dst_hbm, src_hbm, val_hbm, xw_hbm, out_hbm,
               acc, src_b, dst_b, val_b, dst_v, dst_tail_v,
               rows_v, sem):
    c = lax.axis_index("c")
    s = lax.axis_index("s")

    def _fill_rows_zero():
        def _zf(k, carry):
            for j in range(_D // _L):
                rows_v[k, pl.ds(j * _L, _L)] = jnp.zeros((_L,), jnp.float32)
            return carry
        lax.fori_loop(0, _K, _zf, 0)

    def _globalize_src(n, t):
        def _gl(i, carry):
            b = i * _L
            src_b[pl.ds(b, _L)] = src_b[pl.ds(b, _L)] + t * _N
            return carry
        lax.fori_loop(0, n // _L, _gl, 0)

    def _do_chunk(off_e):
        for j in range(_K // _L):
            dst_v[pl.ds(j * _L, _L)] = dst_b[pl.ds(off_e + j * _L, _L)]
        pltpu.async_copy(xw_hbm.at[src_b.at[pl.ds(off_e, _K)]],
                         rows_v, sem).wait()

        def _scale(g, c2):
            val16 = val_b[pl.ds(off_e + g * _L, _L)]
            for l in range(_L):
                bc = _bcast_lane(val16, l)
                k = g * _L + l
                for j in range(_D // _L):
                    rows_v[k, pl.ds(j * _L, _L)] = (
                        rows_v[k, pl.ds(j * _L, _L)] * bc)
            return c2
        lax.fori_loop(0, _K // _L, _scale, 0)

        pltpu.sync_copy(rows_v, acc.at[dst_v], add=True)

    for tt in range(_T // _NC):
        t = c * (_T // _NC) + tt

        _fill_rows_zero()
        r0 = s * _RPW

        def _zero_stripe(rows):
            for q in range(rows // _K):
                pltpu.sync_copy(rows_v, acc.at[pl.ds(r0 + q * _K, _K)])
            rem = rows - (rows // _K) * _K
            if rem:
                pltpu.sync_copy(rows_v.at[pl.ds(0, rem)],
                                acc.at[pl.ds(r0 + (rows // _K) * _K, rem)])

        @pl.when(s < _NS - 1)
        def _():
            _zero_stripe(_RPW)

        @pl.when(s == _NS - 1)
        def _():
            _zero_stripe(_RPW_LAST)

        plsc.subcore_barrier()

        ebase = t * _E + s * _EPW

        def _block(ib, carry):
            eb = ebase + ib * _IB
            pltpu.sync_copy(src_hbm.at[pl.ds(eb, _IB)], src_b)
            pltpu.sync_copy(dst_hbm.at[pl.ds(eb, _IB)], dst_b)
            pltpu.sync_copy(val_hbm.at[pl.ds(eb, _IB)], val_b)
            _globalize_src(_IB, t)

            def _chunk(q, c2):
                _do_chunk(q * _K)
                return c2
            lax.fori_loop(0, _IB // _K, _chunk, 0)
            return carry
        lax.fori_loop(0, _NIB, _block, 0)

        eb = ebase + _NIB * _IB
        pltpu.sync_copy(src_hbm.at[pl.ds(eb, _IBT)], src_b.at[pl.ds(0, _IBT)])
        pltpu.sync_copy(dst_hbm.at[pl.ds(eb, _IBT)], dst_b.at[pl.ds(0, _IBT)])
        pltpu.sync_copy(val_hbm.at[pl.ds(eb, _IBT)], val_b.at[pl.ds(0, _IBT)])
        _globalize_src(_IBT, t)

        def _chunk_t(q, c2):
            _do_chunk(q * _K)
            return c2
        lax.fori_loop(0, _TCH, _chunk_t, 0)

        off_e = _TCH * _K
        for j in range(_TAIL // _L):
            dst_tail_v[pl.ds(j * _L, _L)] = dst_b[pl.ds(off_e + j * _L, _L)]
        pltpu.async_copy(xw_hbm.at[src_b.at[pl.ds(off_e, _TAIL)]],
                         rows_v.at[pl.ds(0, _TAIL)], sem).wait()

        def _scale_tail(g, c2):
            val16 = val_b[pl.ds(off_e + g * _L, _L)]
            for l in range(_L):
                bc = _bcast_lane(val16, l)
                k = g * _L + l
                for j in range(_D // _L):
                    rows_v[k, pl.ds(j * _L, _L)] = (
                        rows_v[k, pl.ds(j * _L, _L)] * bc)
            return c2
        lax.fori_loop(0, _TAIL // _L, _scale_tail, 0)

        pltpu.sync_copy(rows_v.at[pl.ds(0, _TAIL)], acc.at[dst_tail_v],
                        add=True)

        plsc.subcore_barrier()

        obase = t * _N + r0

        @pl.when(s < _NS - 1)
        def _():
            pltpu.sync_copy(acc.at[pl.ds(r0, _RPW)],
                            out_hbm.at[pl.ds(obase, _RPW)])

        @pl.when(s == _NS - 1)
        def _():
            pltpu.sync_copy(acc.at[pl.ds(r0, _RPW_LAST)],
                            out_hbm.at[pl.ds(obase, _RPW_LAST)])


def kernel(adj_indices, adj_values, input, M, W):
    dst = adj_indices[:, 0, :].reshape(-1)
    src = adj_indices[:, 1, :].reshape(-1)
    val = adj_values.reshape(-1)
    x_flat = input.reshape(_T * _N, _D)
    xw = _xw_matmul(x_flat, W)

    mesh = plsc.VectorSubcoreMesh(core_axis_name="c", subcore_axis_name="s")
    spmm = pl.kernel(
        _spmm_body,
        out_type=jax.ShapeDtypeStruct((_T * _N, _D), jnp.float32),
        mesh=mesh,
        scratch_types=[
            pltpu.VMEM_SHARED((_N, _D), jnp.float32),
            pltpu.VMEM((_IB,), jnp.int32),
            pltpu.VMEM((_IB,), jnp.int32),
            pltpu.VMEM((_IB,), jnp.float32),
            pltpu.VMEM((_K,), jnp.int32),
            pltpu.VMEM((_TAIL,), jnp.int32),
            pltpu.VMEM((_K, _D), jnp.float32),
            pltpu.SemaphoreType.DMA,
        ],
    )
    out = spmm(dst, src, val, xw)
    return out.reshape(_T, _N, _D)


# speedup vs baseline: 8.2405x; 1.4728x over previous
import jax
import jax.numpy as jnp
from jax import lax
from jax.experimental import pallas as pl
from jax.experimental.pallas import tpu as pltpu
from jax.experimental.pallas import tpu_sc as plsc

_T, _N, _E, _D = 4, 10000, 320000, 128
_NC, _NS, _L = 2, 16, 16
_EPW = _E // _NS
_K = 128
_IB = 1280
_NIB = _EPW // _IB
_IBT = _EPW - _NIB * _IB
_TCH = _IBT // _K
_TAIL = _IBT - _TCH * _K
_RPW = 632
_RPW_LAST = _N - _RPW * (_NS - 1)


def _bcast_lane(vec16, l):
    idx = jnp.full((_L, 1), l, jnp.int32)
    dn = lax.GatherDimensionNumbers(offset_dims=(), collapsed_slice_dims=(0,),
                                    start_index_map=(0,))
    return lax.gather(vec16, idx, dn, (1,),
                      mode=lax.GatherScatterMode.PROMISE_IN_BOUNDS)


def _mm_body(x_ref, w_ref, o_ref):
    o_ref[...] = jnp.dot(x_ref[...], w_ref[...],
                         preferred_element_type=jnp.float32)


def _xw_matmul(x_flat, W):
    BN = 2000
    return pl.pallas_call(
        _mm_body,
        grid=(x_flat.shape[0] // BN,),
        in_specs=[
            pl.BlockSpec((BN, _D), lambda i: (i, 0)),
            pl.BlockSpec((_D, _D), lambda i: (0, 0)),
        ],
        out_specs=pl.BlockSpec((BN, _D), lambda i: (i, 0)),
        out_shape=jax.ShapeDtypeStruct((x_flat.shape[0], _D), jnp.float32),
    )(x_flat, W)


def _spmm_body(dst_hbm, src_hbm, val_hbm, xw_hbm, out_hbm,
               acc, src_b, dst_b, val_b, dst_v0, dst_v1, dst_tail_v,
               rows0, rows1, sem_g0, sem_g1, sem_s0, sem_s1, sem_i):
    c = lax.axis_index("c")
    s = lax.axis_index("s")

    def _copy_dst(off_e, dvr):
        for j in range(_K // _L):
            dvr[pl.ds(j * _L, _L)] = dst_b[pl.ds(off_e + j * _L, _L)]

    def _scale(rows, off_e):
        def _sc(g, c2):
            val16 = val_b[pl.ds(off_e + g * _L, _L)]
            for l in range(_L):
                bc = _bcast_lane(val16, l)
                k = g * _L + l
                for j in range(_D // _L):
                    rows[k, pl.ds(j * _L, _L)] = rows[k, pl.ds(j * _L, _L)] * bc
            return c2
        lax.fori_loop(0, _K // _L, _sc, 0)

    def _issue_gather(off_e, rows, sem):
        pltpu.async_copy(xw_hbm.at[src_b.at[pl.ds(off_e, _K)]], rows, sem)

    def _wait_gather(rows, sem):
        pltpu.make_async_copy(xw_hbm.at[pl.ds(0, _K)], rows, sem).wait()

    def _issue_scatter(rows, dvr, sem):
        pltpu.async_copy(rows, acc.at[dvr], sem, add=True)

    def _wait_scatter(rows, dvr, sem):
        pltpu.make_async_copy(rows, acc.at[dvr], sem).wait()

    def _load_idx_block(eb, n, t):
        d1 = pltpu.async_copy(src_hbm.at[pl.ds(eb, n)],
                              src_b.at[pl.ds(0, n)], sem_i)
        d2 = pltpu.async_copy(dst_hbm.at[pl.ds(eb, n)],
                              dst_b.at[pl.ds(0, n)], sem_i)
        d3 = pltpu.async_copy(val_hbm.at[pl.ds(eb, n)],
                              val_b.at[pl.ds(0, n)], sem_i)
        d1.wait(); d2.wait(); d3.wait()

        def _gl(i, carry):
            b = i * _L
            src_b[pl.ds(b, _L)] = src_b[pl.ds(b, _L)] + t * _N
            return carry
        lax.fori_loop(0, n // _L, _gl, 0)

    def _run_block(nch, first):
        npair = nch // 2

        _copy_dst(0, dst_v0)
        _issue_gather(0, rows0, sem_g0)

        def _pair(q2, carry):
            off0 = q2 * 2 * _K
            off1 = off0 + _K
            _wait_gather(rows0, sem_g0)

            @pl.when(jnp.logical_not(jnp.logical_and(first, q2 == 0)))
            def _():
                _wait_scatter(rows1, dst_v1, sem_s1)
            _copy_dst(off1, dst_v1)
            _issue_gather(off1, rows1, sem_g1)

            _scale(rows0, off0)
            _issue_scatter(rows0, dst_v0, sem_s0)

            _wait_gather(rows1, sem_g1)
            _scale(rows1, off1)
            _wait_scatter(rows0, dst_v0, sem_s0)

            @pl.when(q2 < npair - 1)
            def _():
                _copy_dst(off1 + _K, dst_v0)
                _issue_gather(off1 + _K, rows0, sem_g0)
            _issue_scatter(rows1, dst_v1, sem_s1)
            return carry
        lax.fori_loop(0, npair, _pair, 0)

    for tt in range(_T // _NC):
        t = c * (_T // _NC) + tt

        def _zf(k, carry):
            for j in range(_D // _L):
                rows0[k, pl.ds(j * _L, _L)] = jnp.zeros((_L,), jnp.float32)
            return carry
        lax.fori_loop(0, _K, _zf, 0)
        r0 = s * _RPW

        def _zero_stripe(rows):
            for q in range(rows // _K):
                pltpu.sync_copy(rows0, acc.at[pl.ds(r0 + q * _K, _K)])
            rem = rows - (rows // _K) * _K
            if rem:
                pltpu.sync_copy(rows0.at[pl.ds(0, rem)],
                                acc.at[pl.ds(r0 + (rows // _K) * _K, rem)])

        @pl.when(s < _NS - 1)
        def _():
            _zero_stripe(_RPW)

        @pl.when(s == _NS - 1)
        def _():
            _zero_stripe(_RPW_LAST)

        plsc.subcore_barrier()

        ebase = t * _E + s * _EPW

        def _block(ib, carry):
            _load_idx_block(ebase + ib * _IB, _IB, t)
            _run_block(_IB // _K, ib == 0)
            return carry
        lax.fori_loop(0, _NIB, _block, 0)

        _load_idx_block(ebase + _NIB * _IB, _IBT, t)
        _run_block(_TCH, jnp.bool_(False))

        off_e = _TCH * _K
        for j in range(_TAIL // _L):
            dst_tail_v[pl.ds(j * _L, _L)] = dst_b[pl.ds(off_e + j * _L, _L)]
        pltpu.async_copy(xw_hbm.at[src_b.at[pl.ds(off_e, _TAIL)]],
                         rows0.at[pl.ds(0, _TAIL)], sem_g0).wait()

        def _scale_tail(g, c2):
            val16 = val_b[pl.ds(off_e + g * _L, _L)]
            for l in range(_L):
                bc = _bcast_lane(val16, l)
                k = g * _L + l
                for j in range(_D // _L):
                    rows0[k, pl.ds(j * _L, _L)] = (
                        rows0[k, pl.ds(j * _L, _L)] * bc)
            return c2
        lax.fori_loop(0, _TAIL // _L, _scale_tail, 0)

        pltpu.sync_copy(rows0.at[pl.ds(0, _TAIL)], acc.at[dst_tail_v],
                        add=True)
        _wait_scatter(rows1, dst_v1, sem_s1)

        plsc.subcore_barrier()

        obase = t * _N + r0

        @pl.when(s < _NS - 1)
        def _():
            pltpu.sync_copy(acc.at[pl.ds(r0, _RPW)],
                            out_hbm.at[pl.ds(obase, _RPW)])

        @pl.when(s == _NS - 1)
        def _():
            pltpu.sync_copy(acc.at[pl.ds(r0, _RPW_LAST)],
                            out_hbm.at[pl.ds(obase, _RPW_LAST)])


def kernel(adj_indices, adj_values, input, M, W):
    dst = adj_indices[:, 0, :].reshape(-1)
    src = adj_indices[:, 1, :].reshape(-1)
    val = adj_values.reshape(-1)
    x_flat = input.reshape(_T * _N, _D)
    xw = _xw_matmul(x_flat, W)

    mesh = plsc.VectorSubcoreMesh(core_axis_name="c", subcore_axis_name="s")
    spmm = pl.kernel(
        _spmm_body,
        out_type=jax.ShapeDtypeStruct((_T * _N, _D), jnp.float32),
        mesh=mesh,
        scratch_types=[
            pltpu.VMEM_SHARED((_N, _D), jnp.float32),
            pltpu.VMEM((_IB,), jnp.int32),
            pltpu.VMEM((_IB,), jnp.int32),
            pltpu.VMEM((_IB,), jnp.float32),
            pltpu.VMEM((_K,), jnp.int32),
            pltpu.VMEM((_K,), jnp.int32),
            pltpu.VMEM((_TAIL,), jnp.int32),
            pltpu.VMEM((_K, _D), jnp.float32),
            pltpu.VMEM((_K, _D), jnp.float32),
            pltpu.SemaphoreType.DMA,
            pltpu.SemaphoreType.DMA,
            pltpu.SemaphoreType.DMA,
            pltpu.SemaphoreType.DMA,
            pltpu.SemaphoreType.DMA,
        ],
    )
    out = spmm(dst, src, val, xw)
    return out.reshape(_T, _N, _D)
